# baseline (speedup 1.0000x reference)
import functools

import jax
import jax.numpy as jnp
from jax import lax
from jax.experimental import pallas as pl
from jax.experimental.pallas import tpu as pltpu
from jax.experimental.pallas import tpu_sc as plsc

N = 10000
D = 128
H = 64
LV = 3
E = 320000

NCORES = 2
NSUB = 16
NTILES = NCORES * NSUB
CH = 128
CPT = 80
EPT = CH * CPT
EPAD = NTILES * EPT
EROWS = EPAD // CH
NPAD = 10240
RPT = NPAD // NSUB
NDUMMY = 16
RB = 1280


def _sc_mesh():
    return plsc.VectorSubcoreMesh(
        core_axis_name="c", subcore_axis_name="s",
        num_cores=NCORES, num_subcores=NSUB)


HR = 5 * NPAD // CH
HRP = 448
SHR = 512


def _k1_hist(idx5f, zrows, sidx):
    def body(idx_ref, z_ref, sidx_ref, out_ref, idx_v, sidx_v, hist5, shacc):
        cid = lax.axis_index("c")
        sid = lax.axis_index("s")
        wid = cid * NSUB + sid
        pltpu.sync_copy(z_ref, hist5)
        pltpu.sync_copy(sidx_ref, sidx_v)
        pltpu.sync_copy(z_ref.at[pl.ds(0, 32)], shacc.at[pl.ds(sid * 32, 32)])
        ones16 = jnp.ones((16,), jnp.float32)
        for h in range(5):
            pltpu.sync_copy(idx_ref.at[pl.ds((h * NTILES + wid) * EPT, EPT)],
                            idx_v)
            off = jnp.full((16,), h * NPAD, jnp.int32)

            def step(j, c, off=off):
                for k in range(4):
                    v = idx_v[pl.ds(j * 64 + k * 16, 16)] + off
                    r = lax.shift_right_logical(v, 7)
                    q = lax.bitwise_and(v, 127)
                    plsc.addupdate_scatter(hist5, [r, q], ones16)
                return c

            lax.fori_loop(0, EPT // 64, step, 0)
        plsc.subcore_barrier()
        for k in range(5):
            pltpu.sync_copy(hist5.at[pl.ds(k * 80, CH)],
                            shacc.at[sidx_v.at[k]], add=True)
        plsc.subcore_barrier()

        @pl.when(sid < 8)
        def _():
            pltpu.sync_copy(shacc.at[pl.ds(sid * 56, 56)],
                            out_ref.at[cid, pl.ds(sid * 56, 56)])

    scratch = [pltpu.VMEM((EPT,), jnp.int32),
               pltpu.VMEM((5, CH), jnp.int32),
               pltpu.VMEM((HRP, CH), jnp.float32),
               pltpu.VMEM_SHARED((SHR, CH), jnp.float32)]
    return pl.kernel(
        body,
        out_type=jax.ShapeDtypeStruct((NCORES, HRP, CH), jnp.float32),
        mesh=_sc_mesh(),
        compiler_params=pltpu.CompilerParams(needs_layout_passes=False),
        scratch_types=scratch,
    )(idx5f, zrows, sidx)


def _k2_scale(xp, xe, dsp, dsr):
    def body(xp_ref, xe_ref, dsp_ref, dsr_ref, yp_ref, ye_ref):
        rsp = lax.rsqrt(jnp.maximum(dsp_ref[0, :] + dsp_ref[1, :], 1.0))
        rse = lax.rsqrt(jnp.maximum(dsr_ref[0, :] + dsr_ref[1, :], 1.0))
        yp_ref[...] = xp_ref[...] * rsp[:, None]
        ye_ref[...] = xe_ref[...] * rse[:, None]

    return pl.pallas_call(
        body,
        out_shape=(jax.ShapeDtypeStruct((N, D), jnp.float32),
                   jax.ShapeDtypeStruct((N, D), jnp.float32)),
    )(xp, xe, dsp, dsr)


def _k3_agg(yp, xg, ye, ei3, zacc):
    def body(yp_ref, xg_ref, ye_ref, ei_ref, z_ref, out_ref,
             src_v, dst_v, rowbuf, acc, gsem, ssem):
        cid = lax.axis_index("c")
        sid = lax.axis_index("s")
        wid = cid * NSUB + sid
        r0 = sid * RPT
        tabs = [yp_ref, xg_ref, ye_ref]
        CPT2 = CPT // 2
        for t in range(3):
            pltpu.sync_copy(z_ref, acc.at[pl.ds(r0, RPT)])
            plsc.subcore_barrier()
            tab = tabs[t]
            for half in range(2):
                hb = wid * CPT + half * CPT2
                pltpu.sync_copy(ei_ref.at[t, 0, pl.ds(hb, CPT2)], src_v)
                pltpu.sync_copy(ei_ref.at[t, 1, pl.ds(hb, CPT2)], dst_v)

                pltpu.async_copy(tab.at[src_v.at[0]], rowbuf.at[0], gsem.at[0])

                def chunk2(i, carry, tab=tab):
                    j0 = i * 2
                    j1 = j0 + 1
                    pltpu.make_async_copy(tab.at[src_v.at[j0]], rowbuf.at[0],
                                          gsem.at[0]).wait()

                    @pl.when(i >= 1)
                    def _():
                        pltpu.make_async_copy(rowbuf.at[1],
                                              acc.at[dst_v.at[j0 - 1]],
                                              ssem.at[1]).wait()

                    pltpu.async_copy(tab.at[src_v.at[j1]], rowbuf.at[1],
                                     gsem.at[1])
                    pltpu.async_copy(rowbuf.at[0], acc.at[dst_v.at[j0]],
                                     ssem.at[0], add=True)
                    pltpu.make_async_copy(tab.at[src_v.at[j1]], rowbuf.at[1],
                                          gsem.at[1]).wait()
                    pltpu.make_async_copy(rowbuf.at[0], acc.at[dst_v.at[j0]],
                                          ssem.at[0]).wait()

                    @pl.when(i + 1 < CPT2 // 2)
                    def _():
                        pltpu.async_copy(tab.at[src_v.at[j0 + 2]],
                                         rowbuf.at[0], gsem.at[0])

                    pltpu.async_copy(rowbuf.at[1], acc.at[dst_v.at[j1]],
                                     ssem.at[1], add=True)
                    return carry

                lax.fori_loop(0, CPT2 // 2, chunk2, 0)
                pltpu.make_async_copy(rowbuf.at[1],
                                      acc.at[dst_v.at[CPT2 - 1]],
                                      ssem.at[1]).wait()
            plsc.subcore_barrier()
            pltpu.sync_copy(acc.at[pl.ds(r0, RPT)],
                            out_ref.at[t, cid, pl.ds(r0, RPT)])
            plsc.subcore_barrier()

    scratch = [pltpu.VMEM((CPT // 2, CH), jnp.int32),
               pltpu.VMEM((CPT // 2, CH), jnp.int32),
               pltpu.VMEM((2, CH, D), jnp.float32),
               pltpu.VMEM_SHARED((NPAD, D), jnp.float32),
               pltpu.SemaphoreType.DMA((2,)),
               pltpu.SemaphoreType.DMA((2,))]
    return pl.kernel(
        body,
        out_type=jax.ShapeDtypeStruct((3, NCORES, NPAD, D), jnp.float32),
        mesh=_sc_mesh(),
        scratch_types=scratch,
    )(yp, xg, ye, ei3, zacc)


def _k4_dense(A, xg, ddp, cnt, ddr,
              W_pert, b_pert, W_sage_l, W_sage_r, b_sage, W_rel, b_rel, S_pool):
    def body(a_ref, xg_ref, ddp_ref, cnt_ref, ddr_ref,
             wp_ref, bp_ref, wl_ref, wr_ref, bs_ref, wq_ref, bq_ref, s_ref,
             out_ref):
        hp = lax.Precision.HIGHEST
        rdp = lax.rsqrt(jnp.maximum(ddp_ref[0, :] + ddp_ref[1, :], 1.0))
        ic = 1.0 / jnp.maximum(cnt_ref[0, :] + cnt_ref[1, :], 1.0)
        rdr = lax.rsqrt(jnp.maximum(ddr_ref[0, :] + ddr_ref[1, :], 1.0))
        Ap = (a_ref[0, 0] + a_ref[0, 1]) * rdp[:, None]
        Am = (a_ref[1, 0] + a_ref[1, 1]) * ic[:, None]
        Ar = (a_ref[2, 0] + a_ref[2, 1]) * rdr[:, None]
        xgb = xg_ref[...]
        for l in range(LV):
            S = s_ref[l]
            wp = jnp.dot(wp_ref[l], S, precision=hp)
            wl = jnp.dot(wl_ref[l], S, precision=hp)
            wr = jnp.dot(wr_ref[l], S, precision=hp)
            wq = jnp.dot(wq_ref[l], S, precision=hp)
            b0 = jnp.dot((bp_ref[l] + bs_ref[l])[None, :], S, precision=hp)
            b1 = jnp.dot(bq_ref[l][None, :], S, precision=hp)
            y0 = (jnp.dot(Ap, wp, precision=hp)
                  + jnp.dot(Am, wl, precision=hp)
                  + jnp.dot(xgb, wr, precision=hp) + b0)
            y1 = jnp.dot(Ar, wq, precision=hp) + b1
            out_ref[l, 0] = y0
            out_ref[l, 1] = y1

    grid = NPAD // RB
    return pl.pallas_call(
        body,
        grid=(grid,),
        in_specs=[
            pl.BlockSpec((LV, NCORES, RB, D), lambda i: (0, 0, i, 0)),
            pl.BlockSpec((RB, D), lambda i: (i, 0)),
            pl.BlockSpec((NCORES, RB), lambda i: (0, i)),
            pl.BlockSpec((NCORES, RB), lambda i: (0, i)),
            pl.BlockSpec((NCORES, RB), lambda i: (0, i)),
            pl.BlockSpec((LV, D, H), lambda i: (0, 0, 0)),
            pl.BlockSpec((LV, H), lambda i: (0, 0)),
            pl.BlockSpec((LV, D, H), lambda i: (0, 0, 0)),
            pl.BlockSpec((LV, D, H), lambda i: (0, 0, 0)),
            pl.BlockSpec((LV, H), lambda i: (0, 0)),
            pl.BlockSpec((LV, D, H), lambda i: (0, 0, 0)),
            pl.BlockSpec((LV, H), lambda i: (0, 0)),
            pl.BlockSpec((LV, H, H), lambda i: (0, 0, 0)),
        ],
        out_specs=pl.BlockSpec((LV, 2, RB, H), lambda i: (0, 0, i, 0)),
        out_shape=jax.ShapeDtypeStruct((LV, 2, N, H), jnp.float32),
    )(A, xg, ddp, cnt, ddr,
      W_pert, b_pert, W_sage_l, W_sage_r, b_sage, W_rel, b_rel, S_pool)


def kernel(x_processo, x_grupo, x_entidade, W_pert, b_pert, W_sage_l,
           W_sage_r, b_sage, W_rel, b_rel, S_pool, ei_pertence, ei_conecta,
           ei_relaciona):
    stripe = (jnp.arange(EPAD - E, dtype=jnp.int32) % NDUMMY).astype(jnp.int32)
    padc = N + stripe
    padg = stripe

    def prep(ei):
        s = jnp.concatenate([ei[0], padg])
        t = jnp.concatenate([ei[1], padc])
        return s, t

    sp, dp = prep(ei_pertence)
    sc_, dc = prep(ei_conecta)
    sr, dr = prep(ei_relaciona)
    spc = jnp.concatenate([ei_pertence[0], padc])
    src = jnp.concatenate([ei_relaciona[0], padc])
    idx5f = jnp.concatenate([spc, dp, dc, src, dr])
    r2 = lambda a: a.reshape(EROWS, CH)
    ei3 = jnp.stack([jnp.stack([r2(sp), r2(dp)]), jnp.stack([r2(sc_), r2(dc)]),
                     jnp.stack([r2(sr), r2(dr)])])

    xp = x_processo
    xg = x_grupo
    xe = x_entidade

    lane = jnp.arange(CH, dtype=jnp.int32)
    sidx = jnp.stack([jnp.where(lane < 80, k * 80 + lane, HRP + (lane % 8))
                      for k in range(5)]).astype(jnp.int32)
    zrows = jnp.zeros((HRP, CH), jnp.float32)
    zacc = jnp.zeros((RPT, D), jnp.float32)

    hist = _k1_hist(idx5f, zrows, sidx)
    h = hist[:, :HR, :].reshape(NCORES, 5, NPAD)
    yp, ye = _k2_scale(xp, xe, h[:, 0, :N], h[:, 3, :N])
    A = _k3_agg(yp, xg, ye, ei3, zacc)
    return _k4_dense(A, xg, h[:, 1], h[:, 2], h[:, 4],
                     W_pert, b_pert, W_sage_l, W_sage_r, b_sage,
                     W_rel, b_rel, S_pool)


# speedup vs baseline: 1.1063x; 1.1063x over previous
import functools

import jax
import jax.numpy as jnp
from jax import lax
from jax.experimental import pallas as pl
from jax.experimental.pallas import tpu as pltpu
from jax.experimental.pallas import tpu_sc as plsc

N = 10000
D = 128
H = 64
LV = 3
E = 320000

NCORES = 2
NSUB = 16
NTILES = NCORES * NSUB
CH = 128
CPT = 80
EPT = CH * CPT
EPAD = NTILES * EPT
EROWS = EPAD // CH
NPAD = 10240
RPT = NPAD // NSUB
NDUMMY = 16
RB = 1280


def _sc_mesh():
    return plsc.VectorSubcoreMesh(
        core_axis_name="c", subcore_axis_name="s",
        num_cores=NCORES, num_subcores=NSUB)


HR = 5 * NPAD // CH
HRP = 448
SHR = 512


def _k1_hist(idx5f, zrows, sidx):
    def body(idx_ref, z_ref, sidx_ref, out_ref, idx_v, sidx_v, hist5, shacc):
        cid = lax.axis_index("c")
        sid = lax.axis_index("s")
        wid = cid * NSUB + sid
        pltpu.sync_copy(z_ref, hist5)
        pltpu.sync_copy(sidx_ref, sidx_v)
        pltpu.sync_copy(z_ref.at[pl.ds(0, 32)], shacc.at[pl.ds(sid * 32, 32)])
        ones16 = jnp.ones((16,), jnp.float32)
        for h in range(5):
            pltpu.sync_copy(idx_ref.at[pl.ds((h * NTILES + wid) * EPT, EPT)],
                            idx_v)
            off = jnp.full((16,), h * NPAD, jnp.int32)

            def step(j, c, off=off):
                for k in range(4):
                    v = idx_v[pl.ds(j * 64 + k * 16, 16)] + off
                    r = lax.shift_right_logical(v, 7)
                    q = lax.bitwise_and(v, 127)
                    plsc.addupdate_scatter(hist5, [r, q], ones16)
                return c

            lax.fori_loop(0, EPT // 64, step, 0)
        plsc.subcore_barrier()
        for k in range(5):
            pltpu.sync_copy(hist5.at[pl.ds(k * 80, CH)],
                            shacc.at[sidx_v.at[k]], add=True)
        plsc.subcore_barrier()

        @pl.when(sid < 8)
        def _():
            pltpu.sync_copy(shacc.at[pl.ds(sid * 56, 56)],
                            out_ref.at[cid, pl.ds(sid * 56, 56)])

    scratch = [pltpu.VMEM((EPT,), jnp.int32),
               pltpu.VMEM((5, CH), jnp.int32),
               pltpu.VMEM((HRP, CH), jnp.float32),
               pltpu.VMEM_SHARED((SHR, CH), jnp.float32)]
    return pl.kernel(
        body,
        out_type=jax.ShapeDtypeStruct((NCORES, HRP, CH), jnp.float32),
        mesh=_sc_mesh(),
        compiler_params=pltpu.CompilerParams(needs_layout_passes=False),
        scratch_types=scratch,
    )(idx5f, zrows, sidx)


def _k2_scale(xp, xe, dsp, dsr):
    def body(xp_ref, xe_ref, dsp_ref, dsr_ref, yp_ref, ye_ref):
        rsp = lax.rsqrt(jnp.maximum(dsp_ref[0, :] + dsp_ref[1, :], 1.0))
        rse = lax.rsqrt(jnp.maximum(dsr_ref[0, :] + dsr_ref[1, :], 1.0))
        yp_ref[...] = xp_ref[...] * rsp[:, None]
        ye_ref[...] = xe_ref[...] * rse[:, None]

    return pl.pallas_call(
        body,
        out_shape=(jax.ShapeDtypeStruct((N, D), jnp.float32),
                   jax.ShapeDtypeStruct((N, D), jnp.float32)),
    )(xp, xe, dsp, dsr)


def _k3_agg(yp, xg, ye, ei3, zacc):
    def body(yp_ref, xg_ref, ye_ref, ei_ref, z_ref, out_ref,
             src_v, dst_v, rowbuf, acc, gsem, ssem):
        cid = lax.axis_index("c")
        sid = lax.axis_index("s")
        wid = cid * NSUB + sid
        r0 = sid * RPT
        tabs = [yp_ref, xg_ref, ye_ref]
        CPT2 = CPT // 2
        for t in range(3):
            pltpu.sync_copy(z_ref, acc.at[pl.ds(r0, RPT)])
            plsc.subcore_barrier()
            tab = tabs[t]
            for half in range(2):
                hb = wid * CPT + half * CPT2
                pltpu.sync_copy(ei_ref.at[t, 0, pl.ds(hb, CPT2)], src_v)
                pltpu.sync_copy(ei_ref.at[t, 1, pl.ds(hb, CPT2)], dst_v)

                pltpu.async_copy(tab.at[src_v.at[0]], rowbuf.at[0], gsem.at[0])

                def chunk(j, carry, tab=tab):
                    cur = lax.rem(j, 2)
                    nxt = 1 - cur

                    @pl.when(j >= 1)
                    def _():
                        pltpu.make_async_copy(rowbuf.at[nxt],
                                              acc.at[dst_v.at[j - 1]],
                                              ssem.at[nxt]).wait()

                    @pl.when(j + 1 < CPT2)
                    def _():
                        pltpu.async_copy(tab.at[src_v.at[j + 1]],
                                         rowbuf.at[nxt], gsem.at[nxt])

                    pltpu.make_async_copy(tab.at[src_v.at[j]], rowbuf.at[cur],
                                          gsem.at[cur]).wait()
                    pltpu.async_copy(rowbuf.at[cur], acc.at[dst_v.at[j]],
                                     ssem.at[cur], add=True)
                    return carry

                lax.fori_loop(0, CPT2, chunk, 0)
                last = (CPT2 - 1) % 2
                pltpu.make_async_copy(rowbuf.at[last],
                                      acc.at[dst_v.at[CPT2 - 1]],
                                      ssem.at[last]).wait()
            plsc.subcore_barrier()
            pltpu.sync_copy(acc.at[pl.ds(r0, RPT)],
                            out_ref.at[t, cid, pl.ds(r0, RPT)])
            plsc.subcore_barrier()

    scratch = [pltpu.VMEM((CPT // 2, CH), jnp.int32),
               pltpu.VMEM((CPT // 2, CH), jnp.int32),
               pltpu.VMEM((2, CH, D), jnp.float32),
               pltpu.VMEM_SHARED((NPAD, D), jnp.float32),
               pltpu.SemaphoreType.DMA((2,)),
               pltpu.SemaphoreType.DMA((2,))]
    return pl.kernel(
        body,
        out_type=jax.ShapeDtypeStruct((3, NCORES, NPAD, D), jnp.float32),
        mesh=_sc_mesh(),
        scratch_types=scratch,
    )(yp, xg, ye, ei3, zacc)


def _k4_dense(A, xg, ddp, cnt, ddr,
              W_pert, b_pert, W_sage_l, W_sage_r, b_sage, W_rel, b_rel, S_pool):
    def body(a_ref, xg_ref, ddp_ref, cnt_ref, ddr_ref,
             wp_ref, bp_ref, wl_ref, wr_ref, bs_ref, wq_ref, bq_ref, s_ref,
             out_ref):
        hp = lax.Precision.HIGHEST
        rdp = lax.rsqrt(jnp.maximum(ddp_ref[0, :] + ddp_ref[1, :], 1.0))
        ic = 1.0 / jnp.maximum(cnt_ref[0, :] + cnt_ref[1, :], 1.0)
        rdr = lax.rsqrt(jnp.maximum(ddr_ref[0, :] + ddr_ref[1, :], 1.0))
        Ap = (a_ref[0, 0] + a_ref[0, 1]) * rdp[:, None]
        Am = (a_ref[1, 0] + a_ref[1, 1]) * ic[:, None]
        Ar = (a_ref[2, 0] + a_ref[2, 1]) * rdr[:, None]
        xgb = xg_ref[...]
        for l in range(LV):
            S = s_ref[l]
            wp = jnp.dot(wp_ref[l], S, precision=hp)
            wl = jnp.dot(wl_ref[l], S, precision=hp)
            wr = jnp.dot(wr_ref[l], S, precision=hp)
            wq = jnp.dot(wq_ref[l], S, precision=hp)
            b0 = jnp.dot((bp_ref[l] + bs_ref[l])[None, :], S, precision=hp)
            b1 = jnp.dot(bq_ref[l][None, :], S, precision=hp)
            y0 = (jnp.dot(Ap, wp, precision=hp)
                  + jnp.dot(Am, wl, precision=hp)
                  + jnp.dot(xgb, wr, precision=hp) + b0)
            y1 = jnp.dot(Ar, wq, precision=hp) + b1
            out_ref[l, 0] = y0
            out_ref[l, 1] = y1

    grid = NPAD // RB
    return pl.pallas_call(
        body,
        grid=(grid,),
        in_specs=[
            pl.BlockSpec((LV, NCORES, RB, D), lambda i: (0, 0, i, 0)),
            pl.BlockSpec((RB, D), lambda i: (i, 0)),
            pl.BlockSpec((NCORES, RB), lambda i: (0, i)),
            pl.BlockSpec((NCORES, RB), lambda i: (0, i)),
            pl.BlockSpec((NCORES, RB), lambda i: (0, i)),
            pl.BlockSpec((LV, D, H), lambda i: (0, 0, 0)),
            pl.BlockSpec((LV, H), lambda i: (0, 0)),
            pl.BlockSpec((LV, D, H), lambda i: (0, 0, 0)),
            pl.BlockSpec((LV, D, H), lambda i: (0, 0, 0)),
            pl.BlockSpec((LV, H), lambda i: (0, 0)),
            pl.BlockSpec((LV, D, H), lambda i: (0, 0, 0)),
            pl.BlockSpec((LV, H), lambda i: (0, 0)),
            pl.BlockSpec((LV, H, H), lambda i: (0, 0, 0)),
        ],
        out_specs=pl.BlockSpec((LV, 2, RB, H), lambda i: (0, 0, i, 0)),
        out_shape=jax.ShapeDtypeStruct((LV, 2, N, H), jnp.float32),
    )(A, xg, ddp, cnt, ddr,
      W_pert, b_pert, W_sage_l, W_sage_r, b_sage, W_rel, b_rel, S_pool)


def kernel(x_processo, x_grupo, x_entidade, W_pert, b_pert, W_sage_l,
           W_sage_r, b_sage, W_rel, b_rel, S_pool, ei_pertence, ei_conecta,
           ei_relaciona):
    stripe = (jnp.arange(EPAD - E, dtype=jnp.int32) % NDUMMY).astype(jnp.int32)
    padc = N + stripe
    padg = stripe

    def prep(ei):
        s = jnp.concatenate([ei[0], padg])
        t = jnp.concatenate([ei[1], padc])
        return s, t

    sp, dp = prep(ei_pertence)
    sc_, dc = prep(ei_conecta)
    sr, dr = prep(ei_relaciona)
    spc = jnp.concatenate([ei_pertence[0], padc])
    src = jnp.concatenate([ei_relaciona[0], padc])
    idx5f = jnp.concatenate([spc, dp, dc, src, dr])
    r2 = lambda a: a.reshape(EROWS, CH)
    ei3 = jnp.stack([jnp.stack([r2(sp), r2(dp)]), jnp.stack([r2(sc_), r2(dc)]),
                     jnp.stack([r2(sr), r2(dr)])])

    xp = x_processo
    xg = x_grupo
    xe = x_entidade

    lane = jnp.arange(CH, dtype=jnp.int32)
    sidx = jnp.stack([jnp.where(lane < 80, k * 80 + lane, HRP + (lane % 8))
                      for k in range(5)]).astype(jnp.int32)
    zrows = jnp.zeros((HRP, CH), jnp.float32)
    zacc = jnp.zeros((RPT, D), jnp.float32)

    hist = _k1_hist(idx5f, zrows, sidx)
    h = hist[:, :HR, :].reshape(NCORES, 5, NPAD)
    yp, ye = _k2_scale(xp, xe, h[:, 0, :N], h[:, 3, :N])
    A = _k3_agg(yp, xg, ye, ei3, zacc)
    return _k4_dense(A, xg, h[:, 1], h[:, 2], h[:, 4],
                     W_pert, b_pert, W_sage_l, W_sage_r, b_sage,
                     W_rel, b_rel, S_pool)
